# baseline (speedup 1.0000x reference)
import functools

import jax
import jax.numpy as jnp
from jax.experimental import pallas as pl
from jax.experimental.pallas import tpu as pltpu

_PW = 256


def _depth_branch_kernel(depth_ref, dflat_ref, hyp_ref, w1_ref, b1_ref,
                         w2_ref, b2_ref, out_ref, feats_ref, idx_ref,
                         hflat_ref, *, H, W, C, D):
    d = pl.program_id(1)
    NP = H * _PW

    @pl.when(d == 0)
    def _compute():
        dfl = dflat_ref[0]
        w1 = w1_ref[...]
        acc1 = jnp.zeros((C, NP), jnp.float32)
        for dy in range(3):
            for dx in range(3):
                t = dy * 3 + dx
                s = dfl[:, dy * _PW + dx: dy * _PW + dx + NP]
                acc1 = acc1 + w1[:, t][:, None] * s
        h = jnp.maximum(acc1 + b1_ref[...].reshape(C, 1), 0.0)
        col = jax.lax.broadcasted_iota(jnp.int32, (1, NP), 1) % _PW
        h = jnp.where(col < W, h, 0.0)
        hflat_ref[...] = jnp.zeros_like(hflat_ref)
        hflat_ref[:, _PW + 1:_PW + 1 + NP] = h

        facc = jnp.zeros((C, NP), jnp.float32)
        for dy in range(3):
            for dx in range(3):
                t = dy * 3 + dx
                hsl = hflat_ref[:, dy * _PW + dx: dy * _PW + dx + NP]
                facc = facc + jax.lax.dot_general(
                    w2_ref[t], hsl, (((1,), (0,)), ((), ())),
                    preferred_element_type=jnp.float32)
        feats = jnp.maximum(facc + b2_ref[...].reshape(C, 1), 0.0)
        feats_ref[...] = feats.reshape(C, H, _PW)[:, :, :W]

        depth = depth_ref[0, 0]
        hyp = hyp_ref[0, 0]
        best = jnp.abs(depth - hyp[0])
        idx = jnp.zeros((H, W), jnp.int32)
        for dd in range(1, D):
            diff = jnp.abs(depth - hyp[dd])
            take = diff < best
            best = jnp.where(take, diff, best)
            idx = jnp.where(take, dd, idx)
        idx_ref[...] = idx

    mask = (idx_ref[...] == d)[None, :, :]
    out_ref[0, :, 0, :, :] = jnp.where(mask, feats_ref[...], 0.0)


def kernel(ref_init_depth, depth_hypotheses, W1, b1, W2, b2):
    B, _, H, W = ref_init_depth.shape
    D = depth_hypotheses.shape[1]
    C = W2.shape[0]
    NFLAT = (H + 3) * _PW

    dpad = jnp.pad(ref_init_depth[:, 0], ((0, 0), (1, 2), (1, _PW - W - 1)))
    dflat = dpad.reshape(B, 1, NFLAT)

    w1r = W1.reshape(C, 9)
    w2r = W2.transpose(2, 3, 0, 1).reshape(9, C, C)
    hyp = depth_hypotheses.reshape(B, 1, D)

    kfn = functools.partial(_depth_branch_kernel, H=H, W=W, C=C, D=D)
    return pl.pallas_call(
        kfn,
        grid=(B, D),
        in_specs=[
            pl.BlockSpec((1, 1, H, W), lambda b, d: (b, 0, 0, 0)),
            pl.BlockSpec((1, 1, NFLAT), lambda b, d: (b, 0, 0)),
            pl.BlockSpec((1, 1, D), lambda b, d: (b, 0, 0)),
            pl.BlockSpec((C, 9), lambda b, d: (0, 0)),
            pl.BlockSpec((1, C), lambda b, d: (0, 0)),
            pl.BlockSpec((9, C, C), lambda b, d: (0, 0, 0)),
            pl.BlockSpec((1, C), lambda b, d: (0, 0)),
        ],
        out_specs=pl.BlockSpec((1, C, 1, H, W), lambda b, d: (b, 0, d, 0, 0)),
        out_shape=jax.ShapeDtypeStruct((B, C, D, H, W), jnp.float32),
        scratch_shapes=[
            pltpu.VMEM((C, H, W), jnp.float32),
            pltpu.VMEM((H, W), jnp.int32),
            pltpu.VMEM((C, NFLAT), jnp.float32),
        ],
        compiler_params=pltpu.CompilerParams(
            dimension_semantics=("parallel", "arbitrary"),
        ),
    )(ref_init_depth, dflat, hyp, w1r, b1.reshape(1, C), w2r,
      b2.reshape(1, C))


# speedup vs baseline: 1.0402x; 1.0402x over previous
import functools

import jax
import jax.numpy as jnp
from jax.experimental import pallas as pl
from jax.experimental.pallas import tpu as pltpu

_PW = 256


def _depth_branch_kernel(depth_ref, dflat_ref, hyp_ref, w1_ref, b1_ref,
                         w2_ref, b2_ref, out_ref, feats_ref, idx_ref,
                         hflat_ref, *, H, W, C, D, PD):
    j = pl.program_id(1)
    NP = H * _PW

    @pl.when(j == 0)
    def _compute():
        dfl = dflat_ref[0]
        w1 = w1_ref[...]
        acc1 = jnp.zeros((C, NP), jnp.float32)
        for dy in range(3):
            for dx in range(3):
                t = dy * 3 + dx
                s = dfl[:, dy * _PW + dx: dy * _PW + dx + NP]
                acc1 = acc1 + w1[:, t][:, None] * s
        h = jnp.maximum(acc1 + b1_ref[...].reshape(C, 1), 0.0)
        col = jax.lax.broadcasted_iota(jnp.int32, (1, NP), 1) % _PW
        h = jnp.where(col < W, h, 0.0)
        hflat_ref[:, :_PW + 1] = jnp.zeros((C, _PW + 1), jnp.float32)
        hflat_ref[:, _PW + 1 + NP:] = jnp.zeros(
            (C, hflat_ref.shape[1] - _PW - 1 - NP), jnp.float32)
        hflat_ref[:, _PW + 1:_PW + 1 + NP] = h

        facc = jnp.zeros((C, NP), jnp.float32)
        for dy in range(3):
            for dx in range(3):
                t = dy * 3 + dx
                hsl = hflat_ref[:, dy * _PW + dx: dy * _PW + dx + NP]
                facc = facc + jax.lax.dot_general(
                    w2_ref[t], hsl, (((1,), (0,)), ((), ())),
                    preferred_element_type=jnp.float32)
        feats = jnp.maximum(facc + b2_ref[...].reshape(C, 1), 0.0)
        feats_ref[...] = feats.reshape(C, H, _PW)[:, :, :W]

        depth = depth_ref[0, 0]
        hyp = hyp_ref[0, 0]
        best = jnp.abs(depth - hyp[0])
        idx = jnp.zeros((H, W), jnp.int32)
        for dd in range(1, D):
            diff = jnp.abs(depth - hyp[dd])
            take = diff < best
            best = jnp.where(take, diff, best)
            idx = jnp.where(take, dd, idx)
        idx_ref[...] = idx

    idx = idx_ref[...]
    feats = feats_ref[...]
    for p in range(PD):
        mask = (idx == j * PD + p)[None, :, :]
        out_ref[0, :, p, :, :] = jnp.where(mask, feats, 0.0)


def kernel(ref_init_depth, depth_hypotheses, W1, b1, W2, b2):
    B, _, H, W = ref_init_depth.shape
    D = depth_hypotheses.shape[1]
    C = W2.shape[0]
    NFLAT = (H + 3) * _PW

    dpad = jnp.pad(ref_init_depth[:, 0], ((0, 0), (1, 2), (1, _PW - W - 1)))
    dflat = dpad.reshape(B, 1, NFLAT)

    w1r = W1.reshape(C, 9)
    w2r = W2.transpose(2, 3, 0, 1).reshape(9, C, C)
    hyp = depth_hypotheses.reshape(B, 1, D)

    PD = 2
    kfn = functools.partial(_depth_branch_kernel, H=H, W=W, C=C, D=D, PD=PD)
    return pl.pallas_call(
        kfn,
        grid=(B, D // PD),
        in_specs=[
            pl.BlockSpec((1, 1, H, W), lambda b, d: (b, 0, 0, 0)),
            pl.BlockSpec((1, 1, NFLAT), lambda b, d: (b, 0, 0)),
            pl.BlockSpec((1, 1, D), lambda b, d: (b, 0, 0)),
            pl.BlockSpec((C, 9), lambda b, d: (0, 0)),
            pl.BlockSpec((1, C), lambda b, d: (0, 0)),
            pl.BlockSpec((9, C, C), lambda b, d: (0, 0, 0)),
            pl.BlockSpec((1, C), lambda b, d: (0, 0)),
        ],
        out_specs=pl.BlockSpec((1, C, PD, H, W), lambda b, d: (b, 0, d, 0, 0)),
        out_shape=jax.ShapeDtypeStruct((B, C, D, H, W), jnp.float32),
        scratch_shapes=[
            pltpu.VMEM((C, H, W), jnp.float32),
            pltpu.VMEM((H, W), jnp.int32),
            pltpu.VMEM((C, NFLAT), jnp.float32),
        ],
        compiler_params=pltpu.CompilerParams(
            dimension_semantics=("parallel", "arbitrary"),
        ),
    )(ref_init_depth, dflat, hyp, w1r, b1.reshape(1, C), w2r,
      b2.reshape(1, C))


# speedup vs baseline: 1.0958x; 1.0535x over previous
import functools

import jax
import jax.numpy as jnp
from jax.experimental import pallas as pl
from jax.experimental.pallas import tpu as pltpu

_PW = 256
_CH = 2048


def _depth_branch_kernel(depth_ref, dflat_ref, hyp_ref, w1_ref, b1_ref,
                         w2_ref, b2_ref, out_ref, feats_ref, idx_ref,
                         hflat_ref, *, H, W, C, D, PD):
    j = pl.program_id(1)
    NP = H * _PW

    @pl.when(j == 0)
    def _compute():
        dfl = dflat_ref[0]
        w1 = w1_ref[...]
        b1c = b1_ref[...].reshape(C, 1)
        b2c = b2_ref[...].reshape(C, 1)
        col = jax.lax.broadcasted_iota(jnp.int32, (1, _CH), 1) % _PW
        vm = col < W

        HOFF = 384
        for n0 in range(0, NP, _CH):
            acc = jnp.broadcast_to(b1c, (C, _CH))
            for dy in range(3):
                for dx in range(3):
                    t = dy * 3 + dx
                    off = dy * _PW + dx + n0
                    acc = acc + w1[:, t][:, None] * dfl[:, off:off + _CH]
            h = jnp.where(vm, jnp.maximum(acc, 0.0), 0.0)
            hflat_ref[:, HOFF + n0:HOFF + n0 + _CH] = h
        hflat_ref[:, :HOFF] = jnp.zeros((C, HOFF), jnp.float32)
        hflat_ref[:, HOFF + NP:] = jnp.zeros(
            (C, hflat_ref.shape[1] - HOFF - NP), jnp.float32)

        RB = _CH // _PW
        for n0 in range(0, NP, _CH):
            facc = None
            for dy in range(3):
                for dx in range(3):
                    t = dy * 3 + dx
                    off = HOFF - _PW - 1 + dy * _PW + dx + n0
                    hsl = hflat_ref[:, off:off + _CH]
                    dres = jax.lax.dot_general(
                        w2_ref[t], hsl, (((1,), (0,)), ((), ())),
                        preferred_element_type=jnp.float32)
                    facc = dres if facc is None else facc + dres
            feats = jnp.maximum(facc + b2c, 0.0)
            y0 = n0 // _PW
            feats_ref[:, y0:y0 + RB, :] = (
                feats.reshape(C, RB, _PW)[:, :, :W])

        hyp = hyp_ref[0, 0]
        YB = 32
        for y0 in range(0, H, YB):
            dch = depth_ref[0, 0, y0:y0 + YB, :]
            best = jnp.abs(dch - hyp[0])
            idx = jnp.zeros((YB, W), jnp.int32)
            for dd in range(1, D):
                diff = jnp.abs(dch - hyp[dd])
                take = diff < best
                best = jnp.where(take, diff, best)
                idx = jnp.where(take, dd, idx)
            idx_ref[y0:y0 + YB, :] = idx

    idx = idx_ref[...]
    feats = feats_ref[...]
    for p in range(PD):
        mask = (idx == j * PD + p)[None, :, :]
        out_ref[0, :, p, :, :] = jnp.where(mask, feats, 0.0)


def kernel(ref_init_depth, depth_hypotheses, W1, b1, W2, b2):
    B, _, H, W = ref_init_depth.shape
    D = depth_hypotheses.shape[1]
    C = W2.shape[0]
    NFLAT = (H + 3) * _PW

    dpad = jnp.pad(ref_init_depth[:, 0], ((0, 0), (1, 2), (1, _PW - W - 1)))
    dflat = dpad.reshape(B, 1, NFLAT)

    w1r = W1.reshape(C, 9)
    w2r = W2.transpose(2, 3, 0, 1).reshape(9, C, C)
    hyp = depth_hypotheses.reshape(B, 1, D)

    PD = 2
    kfn = functools.partial(_depth_branch_kernel, H=H, W=W, C=C, D=D, PD=PD)
    return pl.pallas_call(
        kfn,
        grid=(B, D // PD),
        in_specs=[
            pl.BlockSpec((1, 1, H, W), lambda b, d: (b, 0, 0, 0)),
            pl.BlockSpec((1, 1, NFLAT), lambda b, d: (b, 0, 0)),
            pl.BlockSpec((1, 1, D), lambda b, d: (b, 0, 0)),
            pl.BlockSpec((C, 9), lambda b, d: (0, 0)),
            pl.BlockSpec((1, C), lambda b, d: (0, 0)),
            pl.BlockSpec((9, C, C), lambda b, d: (0, 0, 0)),
            pl.BlockSpec((1, C), lambda b, d: (0, 0)),
        ],
        out_specs=pl.BlockSpec((1, C, PD, H, W), lambda b, d: (b, 0, d, 0, 0)),
        out_shape=jax.ShapeDtypeStruct((B, C, D, H, W), jnp.float32),
        scratch_shapes=[
            pltpu.VMEM((C, H, W), jnp.float32),
            pltpu.VMEM((H, W), jnp.int32),
            pltpu.VMEM((C, (H + 4) * _PW), jnp.float32),
        ],
        compiler_params=pltpu.CompilerParams(
            dimension_semantics=("parallel", "arbitrary"),
        ),
    )(ref_init_depth, dflat, hyp, w1r, b1.reshape(1, C), w2r,
      b2.reshape(1, C))


# speedup vs baseline: 1.1162x; 1.0186x over previous
import functools

import jax
import jax.numpy as jnp
from jax.experimental import pallas as pl
from jax.experimental.pallas import tpu as pltpu

_PW = 256
_CH = 2048


def _depth_branch_kernel(depth_ref, dflat_ref, hyp_ref, w1_ref, b1_ref,
                         w2_ref, b2_ref, out_ref, feats_ref, idx_ref,
                         hflat_ref, *, H, W, C, D, PD):
    j = pl.program_id(1)
    NP = H * _PW

    @pl.when(j == 0)
    def _compute():
        dfl = dflat_ref[0]
        w1 = w1_ref[...]
        b1c = b1_ref[...].reshape(C, 1)
        b2c = b2_ref[...].reshape(C, 1)
        col = jax.lax.broadcasted_iota(jnp.int32, (1, _CH), 1) % _PW
        vm = col < W

        HOFF = 384
        for n0 in range(0, NP, _CH):
            acc = jnp.broadcast_to(b1c, (C, _CH))
            for dy in range(3):
                for dx in range(3):
                    t = dy * 3 + dx
                    off = dy * _PW + dx + n0
                    acc = acc + w1[:, t][:, None] * dfl[:, off:off + _CH]
            h = jnp.where(vm, jnp.maximum(acc, 0.0), 0.0)
            hflat_ref[:, HOFF + n0:HOFF + n0 + _CH] = h
        hflat_ref[:, :HOFF] = jnp.zeros((C, HOFF), jnp.float32)
        hflat_ref[:, HOFF + NP:] = jnp.zeros(
            (C, hflat_ref.shape[1] - HOFF - NP), jnp.float32)

        RB = _CH // _PW
        for n0 in range(0, NP, _CH):
            facc = None
            for dy in range(3):
                for dx in range(3):
                    t = dy * 3 + dx
                    off = HOFF - _PW - 1 + dy * _PW + dx + n0
                    hsl = hflat_ref[:, off:off + _CH]
                    dres = jax.lax.dot_general(
                        w2_ref[t], hsl, (((1,), (0,)), ((), ())),
                        preferred_element_type=jnp.float32)
                    facc = dres if facc is None else facc + dres
            feats = jnp.maximum(facc + b2c, 0.0)
            y0 = n0 // _PW
            feats_ref[:, y0:y0 + RB, :] = (
                feats.reshape(C, RB, _PW)[:, :, :W])

        hyp = hyp_ref[0, 0]
        YB = 32
        for y0 in range(0, H, YB):
            dch = depth_ref[0, 0, y0:y0 + YB, :]
            best = jnp.abs(dch - hyp[0])
            idx = jnp.zeros((YB, W), jnp.int32)
            for dd in range(1, D):
                diff = jnp.abs(dch - hyp[dd])
                take = diff < best
                best = jnp.where(take, diff, best)
                idx = jnp.where(take, dd, idx)
            idx_ref[y0:y0 + YB, :] = idx

    idx = idx_ref[...]
    feats = feats_ref[...]
    for p in range(PD):
        mask = (idx == j * PD + p)[None, :, :]
        out_ref[0, :, p, :, :] = jnp.where(mask, feats, 0.0)


def kernel(ref_init_depth, depth_hypotheses, W1, b1, W2, b2):
    B, _, H, W = ref_init_depth.shape
    D = depth_hypotheses.shape[1]
    C = W2.shape[0]
    NFLAT = (H + 3) * _PW

    dpad = jnp.pad(ref_init_depth[:, 0], ((0, 0), (1, 2), (1, _PW - W - 1)))
    dflat = dpad.reshape(B, 1, NFLAT)

    w1r = W1.reshape(C, 9)
    w2r = W2.transpose(2, 3, 0, 1).reshape(9, C, C)
    hyp = depth_hypotheses.reshape(B, 1, D)

    PD = 4
    kfn = functools.partial(_depth_branch_kernel, H=H, W=W, C=C, D=D, PD=PD)
    return pl.pallas_call(
        kfn,
        grid=(B, D // PD),
        in_specs=[
            pl.BlockSpec((1, 1, H, W), lambda b, d: (b, 0, 0, 0)),
            pl.BlockSpec((1, 1, NFLAT), lambda b, d: (b, 0, 0)),
            pl.BlockSpec((1, 1, D), lambda b, d: (b, 0, 0)),
            pl.BlockSpec((C, 9), lambda b, d: (0, 0)),
            pl.BlockSpec((1, C), lambda b, d: (0, 0)),
            pl.BlockSpec((9, C, C), lambda b, d: (0, 0, 0)),
            pl.BlockSpec((1, C), lambda b, d: (0, 0)),
        ],
        out_specs=pl.BlockSpec((1, C, PD, H, W), lambda b, d: (b, 0, d, 0, 0)),
        out_shape=jax.ShapeDtypeStruct((B, C, D, H, W), jnp.float32),
        scratch_shapes=[
            pltpu.VMEM((C, H, W), jnp.float32),
            pltpu.VMEM((H, W), jnp.int32),
            pltpu.VMEM((C, (H + 4) * _PW), jnp.float32),
        ],
        compiler_params=pltpu.CompilerParams(
            dimension_semantics=("parallel", "arbitrary"),
        ),
    )(ref_init_depth, dflat, hyp, w1r, b1.reshape(1, C), w2r,
      b2.reshape(1, C))


# speedup vs baseline: 1.1702x; 1.0483x over previous
import functools

import jax
import jax.numpy as jnp
from jax.experimental import pallas as pl
from jax.experimental.pallas import tpu as pltpu

_PW = 256
_CH = 1792
_HOFF = 384


def _conv1_piece(dfl, hflat_ref, w1, b1c, vm, c0, c1, C):
    for ci in range(c0, c1):
        n0 = ci * _CH
        acc = jnp.broadcast_to(b1c, (C, _CH))
        for dy in range(3):
            for dx in range(3):
                t = dy * 3 + dx
                off = dy * _PW + dx + n0
                acc = acc + w1[:, t][:, None] * dfl[:, off:off + _CH]
        h = jnp.where(vm, jnp.maximum(acc, 0.0), 0.0)
        hflat_ref[:, _HOFF + n0:_HOFF + n0 + _CH] = h


def _conv2_piece(hflat_ref, feats_ref, w2_ref, b2c, c0, c1, C, W):
    RB = _CH // _PW
    for ci in range(c0, c1):
        n0 = ci * _CH
        facc = None
        for dy in range(3):
            for dx in range(3):
                t = dy * 3 + dx
                off = _HOFF - _PW - 1 + dy * _PW + dx + n0
                dres = jax.lax.dot_general(
                    w2_ref[t], hflat_ref[:, off:off + _CH],
                    (((1,), (0,)), ((), ())),
                    preferred_element_type=jnp.float32)
                facc = dres if facc is None else facc + dres
        feats = jnp.maximum(facc + b2c, 0.0)
        y0 = ci * RB
        feats_ref[:, y0:y0 + RB, :] = feats.reshape(C, RB, _PW)[:, :, :W]


def _argmin_piece(depth_ref, hyp, idx_ref, y0, y1, D, W):
    dch = depth_ref[0, 0, y0:y1, :]
    best = jnp.abs(dch - hyp[0])
    idx = jnp.zeros((y1 - y0, W), jnp.int32)
    for dd in range(1, D):
        diff = jnp.abs(dch - hyp[dd])
        take = diff < best
        best = jnp.where(take, diff, best)
        idx = jnp.where(take, dd, idx)
    idx_ref[y0:y1, :] = idx


def _zero_borders(hflat_ref, C, NP):
    hflat_ref[:, :_HOFF] = jnp.zeros((C, _HOFF), jnp.float32)
    hflat_ref[:, _HOFF + NP:] = jnp.zeros(
        (C, hflat_ref.shape[1] - _HOFF - NP), jnp.float32)


def _depth_branch_kernel(depth_ref, dflat_ref, hyp_ref, depthn_ref,
                         dflatn_ref, hypn_ref, w1_ref, b1_ref, w2_ref,
                         b2_ref, out_ref, feats0_ref, idx0_ref, hflat0_ref,
                         feats1_ref, idx1_ref, hflat1_ref,
                         *, H, W, C, D, PD, B):
    b = pl.program_id(0)
    j = pl.program_id(1)
    NP = H * _PW
    NCH = NP // _CH
    J = D // PD
    CPP = NCH // J
    RPP = H // J

    w1 = w1_ref[...]
    b1c = b1_ref[...].reshape(C, 1)
    b2c = b2_ref[...].reshape(C, 1)
    col = jax.lax.broadcasted_iota(jnp.int32, (1, _CH), 1) % _PW
    vm = col < W

    @pl.when((b == 0) & (j == 0))
    def _prologue():
        dfl = dflat_ref[0]
        hyp = hyp_ref[0, 0]
        _zero_borders(hflat0_ref, C, NP)
        _zero_borders(hflat1_ref, C, NP)
        _conv1_piece(dfl, hflat0_ref, w1, b1c, vm, 0, NCH, C)
        _conv2_piece(hflat0_ref, feats0_ref, w2_ref, b2c, 0, NCH, C, W)
        for k in range(J):
            _argmin_piece(depth_ref, hyp, idx0_ref, k * RPP, (k + 1) * RPP,
                          D, W)

    if B == 2:
        for jj in range(J):
            @pl.when((b == 0) & (j == jj))
            def _piece(jj=jj):
                _conv1_piece(dflatn_ref[0], hflat1_ref, w1, b1c, vm,
                             jj * CPP, (jj + 1) * CPP, C)
                if jj >= 1:
                    _conv2_piece(hflat1_ref, feats1_ref, w2_ref, b2c,
                                 (jj - 1) * CPP, jj * CPP, C, W)
                _argmin_piece(depthn_ref, hypn_ref[0, 0], idx1_ref,
                              jj * RPP, (jj + 1) * RPP, D, W)

        @pl.when((b == 1) & (j == 0))
        def _tail():
            _conv2_piece(hflat1_ref, feats1_ref, w2_ref, b2c,
                         (J - 1) * CPP, NCH, C, W)

    @pl.when(b % 2 == 0)
    def _emit0():
        idx = idx0_ref[...]
        feats = feats0_ref[...]
        for p in range(PD):
            mask = (idx == j * PD + p)[None, :, :]
            out_ref[0, :, p, :, :] = jnp.where(mask, feats, 0.0)

    @pl.when(b % 2 == 1)
    def _emit1():
        idx = idx1_ref[...]
        feats = feats1_ref[...]
        for p in range(PD):
            mask = (idx == j * PD + p)[None, :, :]
            out_ref[0, :, p, :, :] = jnp.where(mask, feats, 0.0)


def kernel(ref_init_depth, depth_hypotheses, W1, b1, W2, b2):
    B, _, H, W = ref_init_depth.shape
    D = depth_hypotheses.shape[1]
    C = W2.shape[0]
    NFLAT = (H + 3) * _PW

    dpad = jnp.pad(ref_init_depth[:, 0], ((0, 0), (1, 2), (1, _PW - W - 1)))
    dflat = dpad.reshape(B, 1, NFLAT)

    w1r = W1.reshape(C, 9)
    w2r = W2.transpose(2, 3, 0, 1).reshape(9, C, C)
    hyp = depth_hypotheses.reshape(B, 1, D)

    PD = 2
    nxt = lambda b, d: (jnp.minimum(b + 1, B - 1), 0, 0)
    nxt4 = lambda b, d: (jnp.minimum(b + 1, B - 1), 0, 0, 0)
    kfn = functools.partial(_depth_branch_kernel, H=H, W=W, C=C, D=D, PD=PD,
                            B=B)
    return pl.pallas_call(
        kfn,
        grid=(B, D // PD),
        in_specs=[
            pl.BlockSpec((1, 1, H, W), lambda b, d: (b, 0, 0, 0)),
            pl.BlockSpec((1, 1, NFLAT), lambda b, d: (b, 0, 0)),
            pl.BlockSpec((1, 1, D), lambda b, d: (b, 0, 0)),
            pl.BlockSpec((1, 1, H, W), nxt4),
            pl.BlockSpec((1, 1, NFLAT), nxt),
            pl.BlockSpec((1, 1, D), nxt),
            pl.BlockSpec((C, 9), lambda b, d: (0, 0)),
            pl.BlockSpec((1, C), lambda b, d: (0, 0)),
            pl.BlockSpec((9, C, C), lambda b, d: (0, 0, 0)),
            pl.BlockSpec((1, C), lambda b, d: (0, 0)),
        ],
        out_specs=pl.BlockSpec((1, C, PD, H, W), lambda b, d: (b, 0, d, 0, 0)),
        out_shape=jax.ShapeDtypeStruct((B, C, D, H, W), jnp.float32),
        scratch_shapes=[
            pltpu.VMEM((C, H, W), jnp.float32),
            pltpu.VMEM((H, W), jnp.int32),
            pltpu.VMEM((C, (H + 4) * _PW), jnp.float32),
            pltpu.VMEM((C, H, W), jnp.float32),
            pltpu.VMEM((H, W), jnp.int32),
            pltpu.VMEM((C, (H + 4) * _PW), jnp.float32),
        ],
        compiler_params=pltpu.CompilerParams(
            dimension_semantics=("arbitrary", "arbitrary"),
        ),
    )(ref_init_depth, dflat, hyp, ref_init_depth, dflat, hyp, w1r,
      b1.reshape(1, C), w2r, b2.reshape(1, C))
